# baseline (speedup 1.0000x reference)
import functools

import jax
import jax.numpy as jnp
from jax import lax
from jax.experimental import pallas as pl
from jax.experimental.pallas import tpu as pltpu
from jax.experimental.pallas import tpu_sc as plsc

_N = 10000
_E = 320000
_D = 128
_L = 16
_NW = 32
_K = 128
_EPW = _E // _NW
_NCH = (_EPW + _K - 1) // _K
_RCH = _N // 80

_mesh = plsc.VectorSubcoreMesh(core_axis_name="c", subcore_axis_name="s")


@functools.partial(
    pl.kernel,
    mesh=_mesh,
    out_type=jax.ShapeDtypeStruct((2, _N, _D), jnp.float32),
    scratch_types=[
        pltpu.VMEM_SHARED((_N, _D), jnp.float32),
        pltpu.VMEM((3, _K), jnp.int32),
        pltpu.VMEM((3, _K), jnp.int32),
        pltpu.VMEM((_K, _D), jnp.float32),
        pltpu.VMEM((_K, _D), jnp.float32),
        pltpu.SemaphoreType.DMA,
        pltpu.SemaphoreType.DMA,
        pltpu.SemaphoreType.DMA,
        pltpu.SemaphoreType.DMA,
    ],
)
def _sc_spmm(table, edges, outp, acc, eb0, eb1, rows0, rows1,
             sg0, sg1, ss0, ss1):
    c = lax.axis_index("c")
    s = lax.axis_index("s")
    w = s * 2 + c
    ebufs = (eb0, eb1)
    bufs = (rows0, rows1)
    sems = (sg0, sg1)
    ssems = (ss0, ss1)

    def _zrow(k, carry):
        for j in range(_D // _L):
            rows0[k, pl.ds(j * _L, _L)] = jnp.zeros((_L,), jnp.float32)
        return carry

    lax.fori_loop(0, 80, _zrow, 0)

    def _zchunk(i, carry):
        cid = s + i * 16

        @pl.when(cid < _RCH)
        def _():
            pltpu.sync_copy(rows0.at[pl.ds(0, 80)], acc.at[pl.ds(cid * 80, 80)])
        return carry

    lax.fori_loop(0, (_RCH + 15) // 16, _zchunk, 0)
    plsc.subcore_barrier()

    pltpu.sync_copy(edges.at[w, 0], eb0)
    pltpu.async_copy(table.at[eb0.at[0]], rows0, sg0)

    def _outer(o, carry):
        for b in range(2):
            ch = o * 2 + b

            @pl.when(ch < _NCH)
            def _():
                nxt = ch + 1

                @pl.when(nxt < _NCH)
                def _():
                    @pl.when(ch >= 1)
                    def _():
                        pltpu.make_async_copy(
                            bufs[1 - b], acc.at[ebufs[1 - b].at[1]],
                            ssems[1 - b]).wait()

                    pltpu.sync_copy(edges.at[w, nxt], ebufs[1 - b])
                    pltpu.async_copy(table.at[ebufs[1 - b].at[0]],
                                     bufs[1 - b], sems[1 - b])

                pltpu.make_async_copy(table.at[ebufs[b].at[0]], bufs[b],
                                      sems[b]).wait()

                def _scale(g, cc):
                    vb = lax.bitcast_convert_type(
                        ebufs[b][2, pl.ds(g * _L, _L)], jnp.float32)
                    for i2 in range(_L):
                        k = g * _L + i2
                        v = vb[i2]
                        for j in range(_D // _L):
                            sl = pl.ds(j * _L, _L)
                            bufs[b][k, sl] = bufs[b][k, sl] * v
                    return cc

                lax.fori_loop(0, _K // _L, _scale, 0)
                pltpu.async_copy(bufs[b], acc.at[ebufs[b].at[1]],
                                 ssems[b], add=True)
        return carry

    lax.fori_loop(0, (_NCH + 1) // 2, _outer, 0)
    pltpu.make_async_copy(bufs[1], acc.at[ebufs[1].at[1]], ssems[1]).wait()
    pltpu.make_async_copy(bufs[0], acc.at[ebufs[0].at[1]], ssems[0]).wait()
    plsc.subcore_barrier()

    def _wchunk(i, carry):
        cid = s + i * 16

        @pl.when(cid < _RCH)
        def _():
            r = cid * 80
            pltpu.sync_copy(acc.at[pl.ds(r, 80)], outp.at[c, pl.ds(r, 80)])
        return carry

    lax.fori_loop(0, (_RCH + 15) // 16, _wchunk, 0)


def _add_body(p_ref, o_ref):
    o_ref[...] = p_ref[0] + p_ref[1]


_ROWS_BLK = 2000


@jax.jit
def _combine(partials):
    return pl.pallas_call(
        _add_body,
        out_shape=jax.ShapeDtypeStruct((_N, _D), jnp.float32),
        grid=(_N // _ROWS_BLK,),
        in_specs=[pl.BlockSpec((2, _ROWS_BLK, _D), lambda i: (0, i, 0))],
        out_specs=pl.BlockSpec((_ROWS_BLK, _D), lambda i: (i, 0)),
    )(partials)


def _pack(x):
    x = x.reshape(_NW, _EPW)
    x = jnp.pad(x, ((0, 0), (0, _NCH * _K - _EPW)))
    return x.reshape(_NW, _NCH, _K)


def kernel(edge_index, edge_values, embeds):
    dst = _pack(edge_index[0].astype(jnp.int32))
    src = _pack(edge_index[1].astype(jnp.int32))
    vbits = _pack(lax.bitcast_convert_type(
        edge_values.astype(jnp.float32), jnp.int32))
    edges = jnp.stack([src, dst, vbits], axis=2)
    partials = _sc_spmm(embeds, edges)
    return _combine(partials)


# speedup vs baseline: 1.3569x; 1.3569x over previous
import functools

import jax
import jax.numpy as jnp
from jax import lax
from jax.experimental import pallas as pl
from jax.experimental.pallas import tpu as pltpu
from jax.experimental.pallas import tpu_sc as plsc

_N = 10000
_E = 320000
_D = 128
_L = 16
_NW = 32
_K = 80
_EPW = _E // _NW
_NCH = _EPW // _K
_RCH = _N // 80

_mesh = plsc.VectorSubcoreMesh(core_axis_name="c", subcore_axis_name="s")


@functools.partial(
    pl.kernel,
    mesh=_mesh,
    out_type=jax.ShapeDtypeStruct((2, _N, _D), jnp.float32),
    scratch_types=[
        pltpu.VMEM_SHARED((_N, _D), jnp.float32),
        pltpu.VMEM((3, _K), jnp.int32),
        pltpu.VMEM((3, _K), jnp.int32),
        pltpu.VMEM((_K, _D), jnp.float32),
        pltpu.VMEM((_K, _D), jnp.float32),
        pltpu.SemaphoreType.DMA,
        pltpu.SemaphoreType.DMA,
        pltpu.SemaphoreType.DMA,
        pltpu.SemaphoreType.DMA,
    ],
)
def _sc_spmm(table, edges, outp, acc, eb0, eb1, rows0, rows1,
             sg0, sg1, ss0, ss1):
    c = lax.axis_index("c")
    s = lax.axis_index("s")
    w = s * 2 + c
    ebufs = (eb0, eb1)
    bufs = (rows0, rows1)
    sems = (sg0, sg1)
    ssems = (ss0, ss1)

    def _zrow(k, carry):
        for j in range(_D // _L):
            rows0[k, pl.ds(j * _L, _L)] = jnp.zeros((_L,), jnp.float32)
        return carry

    lax.fori_loop(0, 80, _zrow, 0)

    def _zchunk(i, carry):
        cid = s + i * 16

        @pl.when(cid < _RCH)
        def _():
            pltpu.sync_copy(rows0.at[pl.ds(0, 80)], acc.at[pl.ds(cid * 80, 80)])
        return carry

    lax.fori_loop(0, (_RCH + 15) // 16, _zchunk, 0)
    plsc.subcore_barrier()

    pltpu.sync_copy(edges.at[w, 0], eb0)
    pltpu.async_copy(table.at[eb0.at[0]], rows0, sg0)

    def _outer(o, carry):
        for b in range(2):
            ch = o * 2 + b

            @pl.when(ch < _NCH)
            def _():
                nxt = ch + 1

                @pl.when(nxt < _NCH)
                def _():
                    @pl.when(ch >= 1)
                    def _():
                        pltpu.make_async_copy(
                            bufs[1 - b], acc.at[ebufs[1 - b].at[1]],
                            ssems[1 - b]).wait()

                    pltpu.sync_copy(edges.at[w, nxt], ebufs[1 - b])
                    pltpu.async_copy(table.at[ebufs[1 - b].at[0]],
                                     bufs[1 - b], sems[1 - b])

                pltpu.make_async_copy(table.at[ebufs[b].at[0]], bufs[b],
                                      sems[b]).wait()

                def _scale(g, cc):
                    vb = lax.bitcast_convert_type(
                        ebufs[b][2, pl.ds(g * _L, _L)], jnp.float32)
                    for i2 in range(_L):
                        k = g * _L + i2
                        v = vb[i2]
                        for j in range(_D // _L):
                            sl = pl.ds(j * _L, _L)
                            bufs[b][k, sl] = bufs[b][k, sl] * v
                    return cc

                lax.fori_loop(0, _K // _L, _scale, 0)
                pltpu.async_copy(bufs[b], acc.at[ebufs[b].at[1]],
                                 ssems[b], add=True)
        return carry

    lax.fori_loop(0, (_NCH + 1) // 2, _outer, 0)
    pltpu.make_async_copy(bufs[1], acc.at[ebufs[1].at[1]], ssems[1]).wait()
    pltpu.make_async_copy(bufs[0], acc.at[ebufs[0].at[1]], ssems[0]).wait()
    plsc.subcore_barrier()

    def _wchunk(i, carry):
        cid = s + i * 16

        @pl.when(cid < _RCH)
        def _():
            r = cid * 80
            pltpu.sync_copy(acc.at[pl.ds(r, 80)], outp.at[c, pl.ds(r, 80)])
        return carry

    lax.fori_loop(0, (_RCH + 15) // 16, _wchunk, 0)


def _add_body(p_ref, o_ref):
    o_ref[...] = p_ref[0] + p_ref[1]


_ROWS_BLK = 2000


@jax.jit
def _combine(partials):
    return pl.pallas_call(
        _add_body,
        out_shape=jax.ShapeDtypeStruct((_N, _D), jnp.float32),
        grid=(_N // _ROWS_BLK,),
        in_specs=[pl.BlockSpec((2, _ROWS_BLK, _D), lambda i: (0, i, 0))],
        out_specs=pl.BlockSpec((_ROWS_BLK, _D), lambda i: (i, 0)),
    )(partials)


def _pack(x):
    return x.reshape(_NW, _NCH, _K)


def kernel(edge_index, edge_values, embeds):
    dst = _pack(edge_index[0].astype(jnp.int32))
    src = _pack(edge_index[1].astype(jnp.int32))
    vbits = _pack(lax.bitcast_convert_type(
        edge_values.astype(jnp.float32), jnp.int32))
    edges = jnp.stack([src, dst, vbits], axis=2)
    partials = _sc_spmm(embeds, edges)
    return _combine(partials)


# speedup vs baseline: 1.3829x; 1.0192x over previous
import functools

import jax
import jax.numpy as jnp
from jax import lax
from jax.experimental import pallas as pl
from jax.experimental.pallas import tpu as pltpu
from jax.experimental.pallas import tpu_sc as plsc

_N = 10000
_E = 320000
_D = 128
_L = 16
_NW = 32
_K = 80
_EPW = _E // _NW
_NCH = _EPW // _K
_RCH = _N // 80

_mesh = plsc.VectorSubcoreMesh(core_axis_name="c", subcore_axis_name="s")


@functools.partial(
    pl.kernel,
    mesh=_mesh,
    out_type=jax.ShapeDtypeStruct((2, _N, _D), jnp.float32),
    scratch_types=[
        pltpu.VMEM_SHARED((_N, _D), jnp.float32),
        pltpu.VMEM((2, 3, _K), jnp.int32),
        pltpu.VMEM((2, 3, _K), jnp.int32),
        pltpu.VMEM((_K, _D), jnp.float32),
        pltpu.VMEM((_K, _D), jnp.float32),
        pltpu.SemaphoreType.DMA,
        pltpu.SemaphoreType.DMA,
        pltpu.SemaphoreType.DMA,
        pltpu.SemaphoreType.DMA,
    ],
)
def _sc_spmm(table, edges, outp, acc, pb0, pb1, rows0, rows1,
             sg0, sg1, ss0, ss1):
    c = lax.axis_index("c")
    s = lax.axis_index("s")
    w = s * 2 + c
    pbufs = (pb0, pb1)
    bufs = (rows0, rows1)
    sems = (sg0, sg1)
    ssems = (ss0, ss1)

    def _zrow(k, carry):
        for j in range(_D // _L):
            rows0[k, pl.ds(j * _L, _L)] = jnp.zeros((_L,), jnp.float32)
        return carry

    lax.fori_loop(0, 80, _zrow, 0)

    def _zchunk(i, carry):
        cid = s + i * 16

        @pl.when(cid < _RCH)
        def _():
            pltpu.sync_copy(rows0.at[pl.ds(0, 80)], acc.at[pl.ds(cid * 80, 80)])
        return carry

    lax.fori_loop(0, (_RCH + 15) // 16, _zchunk, 0)
    plsc.subcore_barrier()

    pltpu.sync_copy(edges.at[w, pl.ds(0, 2)], pb0)
    pltpu.async_copy(table.at[pb0.at[0, 0]], rows0, sg0)

    def _outer(o, carry):
        for u in range(4):
            ch = o * 4 + u
            b = u % 2
            pb = pbufs[u // 2]

            @pl.when(ch < _NCH)
            def _():
                nxt = ch + 1

                @pl.when(nxt < _NCH)
                def _():
                    @pl.when(ch >= 1)
                    def _():
                        pltpu.make_async_copy(
                            bufs[1 - b],
                            acc.at[pbufs[(u + 1) % 4 // 2].at[1 - b, 1]],
                            ssems[1 - b]).wait()

                    if u % 2 == 1:
                        pltpu.sync_copy(edges.at[w, pl.ds(nxt, 2)],
                                        pbufs[1 - u // 2])
                    pltpu.async_copy(
                        table.at[pbufs[(u + 1) % 4 // 2].at[1 - b, 0]],
                        bufs[1 - b], sems[1 - b])

                pltpu.make_async_copy(table.at[pb.at[b, 0]], bufs[b],
                                      sems[b]).wait()

                def _scale(g, cc):
                    vb = lax.bitcast_convert_type(
                        pb[b, 2, pl.ds(g * _L, _L)], jnp.float32)
                    for i2 in range(_L):
                        k = g * _L + i2
                        v = vb[i2]
                        for j in range(_D // _L):
                            sl = pl.ds(j * _L, _L)
                            bufs[b][k, sl] = bufs[b][k, sl] * v
                    return cc

                lax.fori_loop(0, _K // _L, _scale, 0)
                pltpu.async_copy(bufs[b], acc.at[pb.at[b, 1]],
                                 ssems[b], add=True)
        return carry

    lax.fori_loop(0, (_NCH + 3) // 4, _outer, 0)
    pltpu.make_async_copy(bufs[1], acc.at[pb0.at[1, 1]], ssems[1]).wait()
    pltpu.make_async_copy(bufs[0], acc.at[pb0.at[0, 1]], ssems[0]).wait()
    plsc.subcore_barrier()

    def _wchunk(i, carry):
        cid = s + i * 16

        @pl.when(cid < _RCH)
        def _():
            r = cid * 80
            pltpu.sync_copy(acc.at[pl.ds(r, 80)], outp.at[c, pl.ds(r, 80)])
        return carry

    lax.fori_loop(0, (_RCH + 15) // 16, _wchunk, 0)


def _add_body(p_ref, o_ref):
    o_ref[...] = p_ref[0] + p_ref[1]


_ROWS_BLK = 2000


@jax.jit
def _combine(partials):
    return pl.pallas_call(
        _add_body,
        out_shape=jax.ShapeDtypeStruct((_N, _D), jnp.float32),
        grid=(_N // _ROWS_BLK,),
        in_specs=[pl.BlockSpec((2, _ROWS_BLK, _D), lambda i: (0, i, 0))],
        out_specs=pl.BlockSpec((_ROWS_BLK, _D), lambda i: (i, 0)),
    )(partials)


def _pack(x):
    return x.reshape(_NW, _NCH, _K)


def kernel(edge_index, edge_values, embeds):
    dst = _pack(edge_index[0].astype(jnp.int32))
    src = _pack(edge_index[1].astype(jnp.int32))
    vbits = _pack(lax.bitcast_convert_type(
        edge_values.astype(jnp.float32), jnp.int32))
    edges = jnp.stack([src, dst, vbits], axis=2)
    edges = jnp.pad(edges, ((0, 0), (0, 1), (0, 0), (0, 0)))
    partials = _sc_spmm(embeds, edges)
    return _combine(partials)
